# Initial kernel scaffold; baseline (speedup 1.0000x reference)
import functools

import jax
import jax.numpy as jnp
from jax import lax
from jax.experimental import pallas as pl
from jax.experimental.pallas import tpu as pltpu
from jax.experimental.pallas import tpu_sc as plsc

N_NODES = 10000
N_EDGES = 320000
D = 128
NUM_SEG = 64

NC = 2
NS = 16
NW = NC * NS
EPW = N_EDGES // NW
CH = 80
NCH = EPW // CH
RPT = N_NODES // NS
CW = 16

BLK = 1000
NBLK = N_NODES // BLK


def _ln_body(x_ref, w_ref, b_ref, o_ref):
    xv = x_ref[...]
    mu = jnp.mean(xv, axis=-1, keepdims=True)
    var = jnp.mean((xv - mu) ** 2, axis=-1, keepdims=True)
    o_ref[...] = (xv - mu) * lax.rsqrt(var + 1e-5) * w_ref[...] + b_ref[...]


def _layernorm(x, w, b):
    return pl.pallas_call(
        _ln_body,
        grid=(NBLK,),
        in_specs=[
            pl.BlockSpec((BLK, D), lambda i: (i, 0)),
            pl.BlockSpec((1, D), lambda i: (0, 0)),
            pl.BlockSpec((1, D), lambda i: (0, 0)),
        ],
        out_specs=pl.BlockSpec((BLK, D), lambda i: (i, 0)),
        out_shape=jax.ShapeDtypeStruct((N_NODES, D), jnp.float32),
    )(x, w.reshape(1, D), b.reshape(1, D))


def _sc_agg_body(h_hbm, src_hbm, dst_hbm, zc_hbm, zn_hbm, ones_hbm,
                 agg_out, cnt_out, agg_sh, cnt_sh, sidx, didx, rows, ones_v,
                 sem):
    c = lax.axis_index("c")
    s = lax.axis_index("s")
    wid = c * NS + s
    r0 = s * RPT
    pltpu.sync_copy(zc_hbm.at[pl.ds(r0, RPT)], agg_sh.at[pl.ds(r0, RPT)])
    pltpu.sync_copy(zn_hbm.at[pl.ds(r0, RPT)], cnt_sh.at[pl.ds(r0, RPT)])
    pltpu.sync_copy(ones_hbm, ones_v)
    plsc.subcore_barrier()

    def body(ci, carry):
        base = wid * EPW + ci * CH
        pltpu.sync_copy(src_hbm.at[pl.ds(base, CH)], sidx)
        pltpu.async_copy(h_hbm.at[sidx], rows, sem).wait()
        pltpu.sync_copy(dst_hbm.at[pl.ds(base, CH)], didx)
        pltpu.sync_copy(rows, agg_sh.at[didx], add=True)
        pltpu.sync_copy(ones_v, cnt_sh.at[didx], add=True)
        return carry

    lax.fori_loop(0, NCH, body, 0)
    plsc.subcore_barrier()
    pltpu.sync_copy(agg_sh.at[pl.ds(r0, RPT)], agg_out.at[c, pl.ds(r0, RPT)])
    pltpu.sync_copy(cnt_sh.at[pl.ds(r0, RPT)], cnt_out.at[c, pl.ds(r0, RPT)])


_sc_agg = functools.partial(
    pl.kernel,
    out_type=[
        jax.ShapeDtypeStruct((NC, N_NODES, D), jnp.float32),
        jax.ShapeDtypeStruct((NC, N_NODES, CW), jnp.float32),
    ],
    mesh=plsc.VectorSubcoreMesh(core_axis_name="c", subcore_axis_name="s"),
    scratch_types=[
        pltpu.VMEM_SHARED((N_NODES, D), jnp.float32),
        pltpu.VMEM_SHARED((N_NODES, CW), jnp.float32),
        pltpu.VMEM((CH,), jnp.int32),
        pltpu.VMEM((CH,), jnp.int32),
        pltpu.VMEM((CH, D), jnp.float32),
        pltpu.VMEM((CH, CW), jnp.float32),
        pltpu.SemaphoreType.DMA,
    ],
)(_sc_agg_body)


def _combine_body(aggp_ref, cntp_ref, h_ref, seg_ref, wl_ref, wr_ref, bl_ref,
                  n1w_ref, n1b_ref, o_ref, acc, cacc):
    i = pl.program_id(0)
    agg = aggp_ref[0] + aggp_ref[1]
    cnt = cntp_ref[0, :, 0:1] + cntp_ref[1, :, 0:1]
    mean_agg = agg / jnp.maximum(cnt, 1.0)
    h = h_ref[...]
    h2 = (
        lax.dot_general(mean_agg, wl_ref[...], (((1,), (1,)), ((), ())),
                        preferred_element_type=jnp.float32)
        + bl_ref[...]
        + lax.dot_general(h, wr_ref[...], (((1,), (1,)), ((), ())),
                          preferred_element_type=jnp.float32)
    )
    h2 = jnp.maximum(h2, 0.0)
    mu = jnp.mean(h2, axis=-1, keepdims=True)
    var = jnp.mean((h2 - mu) ** 2, axis=-1, keepdims=True)
    h2 = (h2 - mu) * lax.rsqrt(var + 1e-5) * n1w_ref[...] + n1b_ref[...]

    seg = seg_ref[0, 0, :]
    onehot = (seg[:, None]
              == lax.broadcasted_iota(jnp.int32, (BLK, NUM_SEG), 1)
              ).astype(jnp.float32)

    @pl.when(i == 0)
    def _():
        acc[...] = jnp.zeros_like(acc)
        cacc[...] = jnp.zeros_like(cacc)

    acc[...] += lax.dot_general(onehot, h2, (((0,), (0,)), ((), ())),
                                preferred_element_type=jnp.float32)
    cacc[...] += lax.dot_general(onehot, jnp.ones_like(h2),
                                 (((0,), (0,)), ((), ())),
                                 preferred_element_type=jnp.float32)

    @pl.when(i == NBLK - 1)
    def _():
        o_ref[...] = acc[...] / jnp.maximum(cacc[...], 1.0)


def _combine(aggp, cntp, h, seg3, W_l, W_r, b_l, n1w, n1b):
    return pl.pallas_call(
        _combine_body,
        grid=(NBLK,),
        in_specs=[
            pl.BlockSpec((NC, BLK, D), lambda i: (0, i, 0)),
            pl.BlockSpec((NC, BLK, CW), lambda i: (0, i, 0)),
            pl.BlockSpec((BLK, D), lambda i: (i, 0)),
            pl.BlockSpec((1, 1, BLK), lambda i: (i, 0, 0)),
            pl.BlockSpec((D, D), lambda i: (0, 0)),
            pl.BlockSpec((D, D), lambda i: (0, 0)),
            pl.BlockSpec((1, D), lambda i: (0, 0)),
            pl.BlockSpec((1, D), lambda i: (0, 0)),
            pl.BlockSpec((1, D), lambda i: (0, 0)),
        ],
        out_specs=pl.BlockSpec((NUM_SEG, D), lambda i: (0, 0)),
        out_shape=jax.ShapeDtypeStruct((NUM_SEG, D), jnp.float32),
        scratch_shapes=[
            pltpu.VMEM((NUM_SEG, D), jnp.float32),
            pltpu.VMEM((NUM_SEG, D), jnp.float32),
        ],
    )(aggp, cntp, h, seg3, W_l, W_r, b_l.reshape(1, D), n1w.reshape(1, D),
      n1b.reshape(1, D))


def kernel(x, edge_patch, superpixel_attri, W_l, b_l, W_r,
           norm_w, norm_b, norm1_w, norm1_b):
    src = edge_patch[0].astype(jnp.int32)
    dst = edge_patch[1].astype(jnp.int32)
    seg3 = superpixel_attri.astype(jnp.int32).reshape(NBLK, 1, BLK)
    zc = jnp.zeros((N_NODES, D), jnp.float32)
    zn = jnp.zeros((N_NODES, CW), jnp.float32)
    ones = jnp.ones((CH, CW), jnp.float32)

    h = _layernorm(x, norm_w, norm_b)
    aggp, cntp = _sc_agg(h, src, dst, zc, zn, ones)
    return _combine(aggp, cntp, h, seg3, W_l, W_r, b_l, norm1_w, norm1_b)


# speedup vs baseline: 6.1024x; 6.1024x over previous
import functools

import jax
import jax.numpy as jnp
from jax import lax
from jax.experimental import pallas as pl
from jax.experimental.pallas import tpu as pltpu
from jax.experimental.pallas import tpu_sc as plsc

N_NODES = 10000
N_EDGES = 320000
D = 128
NUM_SEG = 64

NC = 2
NS = 16
NW = NC * NS
EPW = N_EDGES // NW
CH = 80
NCH = EPW // CH
NP = 10240
RPT = NP // NS
CW = 16

BLK = 1000
NBLK = N_NODES // BLK


def _ln_body(x_ref, w_ref, b_ref, o_ref):
    xv = x_ref[...]
    mu = jnp.mean(xv, axis=-1, keepdims=True)
    var = jnp.mean((xv - mu) ** 2, axis=-1, keepdims=True)
    o_ref[...] = (xv - mu) * lax.rsqrt(var + 1e-5) * w_ref[...] + b_ref[...]


def _layernorm(x, w, b):
    return pl.pallas_call(
        _ln_body,
        grid=(NBLK,),
        in_specs=[
            pl.BlockSpec((BLK, D), lambda i: (i, 0)),
            pl.BlockSpec((1, D), lambda i: (0, 0)),
            pl.BlockSpec((1, D), lambda i: (0, 0)),
        ],
        out_specs=pl.BlockSpec((BLK, D), lambda i: (i, 0)),
        out_shape=jax.ShapeDtypeStruct((N_NODES, D), jnp.float32),
    )(x, w.reshape(1, D), b.reshape(1, D))


def _sc_agg_body(h_hbm, src_hbm, dst_hbm, zc_hbm, zn_hbm, ones_hbm,
                 agg_out, cnt_out, agg_sh, cnt_sh, sidx, didx, rows, ones_v,
                 sem):
    c = lax.axis_index("c")
    s = lax.axis_index("s")
    wid = c * NS + s
    r0 = s * RPT
    pltpu.sync_copy(zc_hbm, rows)
    pltpu.sync_copy(zn_hbm, ones_v)
    for j in range(RPT // CH):
        pltpu.sync_copy(rows, agg_sh.at[pl.ds(r0 + j * CH, CH)])
        pltpu.sync_copy(ones_v, cnt_sh.at[pl.ds(r0 + j * CH, CH)])
    pltpu.sync_copy(ones_hbm, ones_v)
    plsc.subcore_barrier()

    def body(ci, carry):
        base = wid * EPW + ci * CH
        pltpu.sync_copy(src_hbm.at[pl.ds(base, CH)], sidx)
        pltpu.async_copy(h_hbm.at[sidx], rows, sem).wait()
        pltpu.sync_copy(dst_hbm.at[pl.ds(base, CH)], didx)
        pltpu.sync_copy(rows, agg_sh.at[didx], add=True)
        pltpu.sync_copy(ones_v, cnt_sh.at[didx], add=True)
        return carry

    lax.fori_loop(0, NCH, body, 0)
    plsc.subcore_barrier()
    for j in range(RPT // CH):
        pltpu.sync_copy(agg_sh.at[pl.ds(r0 + j * CH, CH)], rows)
        pltpu.sync_copy(rows, agg_out.at[c, pl.ds(r0 + j * CH, CH)])
        pltpu.sync_copy(cnt_sh.at[pl.ds(r0 + j * CH, CH)], ones_v)
        pltpu.sync_copy(ones_v, cnt_out.at[c, pl.ds(r0 + j * CH, CH)])


@functools.cache
def _sc_agg_kernel():
    return pl.kernel(
        _sc_agg_body,
        out_type=[
            jax.ShapeDtypeStruct((NC, NP, D), jnp.float32),
            jax.ShapeDtypeStruct((NC, NP, CW), jnp.float32),
        ],
        mesh=plsc.VectorSubcoreMesh(core_axis_name="c", subcore_axis_name="s",
                                    num_cores=NC, num_subcores=NS),
        scratch_types=[
            pltpu.VMEM_SHARED((NP, D), jnp.float32),
            pltpu.VMEM_SHARED((NP, CW), jnp.float32),
            pltpu.VMEM((CH,), jnp.int32),
            pltpu.VMEM((CH,), jnp.int32),
            pltpu.VMEM((CH, D), jnp.float32),
            pltpu.VMEM((CH, CW), jnp.float32),
            pltpu.SemaphoreType.DMA,
        ],
        compiler_params=pltpu.CompilerParams(use_tc_tiling_on_sc=False),
    )


def _combine_body(aggp_ref, cntp_ref, h_ref, seg_ref, wl_ref, wr_ref, bl_ref,
                  n1w_ref, n1b_ref, o_ref, acc, cacc):
    i = pl.program_id(0)
    agg = aggp_ref[0] + aggp_ref[1]
    cnt = cntp_ref[0, :, 0:1] + cntp_ref[1, :, 0:1]
    mean_agg = agg / jnp.maximum(cnt, 1.0)
    h = h_ref[...]
    h2 = (
        lax.dot_general(mean_agg, wl_ref[...], (((1,), (1,)), ((), ())),
                        preferred_element_type=jnp.float32)
        + bl_ref[...]
        + lax.dot_general(h, wr_ref[...], (((1,), (1,)), ((), ())),
                          preferred_element_type=jnp.float32)
    )
    h2 = jnp.maximum(h2, 0.0)
    mu = jnp.mean(h2, axis=-1, keepdims=True)
    var = jnp.mean((h2 - mu) ** 2, axis=-1, keepdims=True)
    h2 = (h2 - mu) * lax.rsqrt(var + 1e-5) * n1w_ref[...] + n1b_ref[...]

    seg = seg_ref[0, 0, :]
    onehot = (seg[:, None]
              == lax.broadcasted_iota(jnp.int32, (BLK, NUM_SEG), 1)
              ).astype(jnp.float32)

    @pl.when(i == 0)
    def _():
        acc[...] = jnp.zeros_like(acc)
        cacc[...] = jnp.zeros_like(cacc)

    acc[...] += lax.dot_general(onehot, h2, (((0,), (0,)), ((), ())),
                                preferred_element_type=jnp.float32)
    cacc[...] += lax.dot_general(onehot, jnp.ones_like(h2),
                                 (((0,), (0,)), ((), ())),
                                 preferred_element_type=jnp.float32)

    @pl.when(i == NBLK - 1)
    def _():
        o_ref[...] = acc[...] / jnp.maximum(cacc[...], 1.0)


def _combine(aggp, cntp, h, seg3, W_l, W_r, b_l, n1w, n1b):
    return pl.pallas_call(
        _combine_body,
        grid=(NBLK,),
        in_specs=[
            pl.BlockSpec((NC, BLK, D), lambda i: (0, i, 0)),
            pl.BlockSpec((NC, BLK, CW), lambda i: (0, i, 0)),
            pl.BlockSpec((BLK, D), lambda i: (i, 0)),
            pl.BlockSpec((1, 1, BLK), lambda i: (i, 0, 0)),
            pl.BlockSpec((D, D), lambda i: (0, 0)),
            pl.BlockSpec((D, D), lambda i: (0, 0)),
            pl.BlockSpec((1, D), lambda i: (0, 0)),
            pl.BlockSpec((1, D), lambda i: (0, 0)),
            pl.BlockSpec((1, D), lambda i: (0, 0)),
        ],
        out_specs=pl.BlockSpec((NUM_SEG, D), lambda i: (0, 0)),
        out_shape=jax.ShapeDtypeStruct((NUM_SEG, D), jnp.float32),
        scratch_shapes=[
            pltpu.VMEM((NUM_SEG, D), jnp.float32),
            pltpu.VMEM((NUM_SEG, D), jnp.float32),
        ],
    )(aggp, cntp, h, seg3, W_l, W_r, b_l.reshape(1, D), n1w.reshape(1, D),
      n1b.reshape(1, D))


def kernel(x, edge_patch, superpixel_attri, W_l, b_l, W_r,
           norm_w, norm_b, norm1_w, norm1_b):
    src = edge_patch[0].astype(jnp.int32)
    dst = edge_patch[1].astype(jnp.int32)
    seg3 = superpixel_attri.astype(jnp.int32).reshape(NBLK, 1, BLK)
    zc = jnp.zeros((CH, D), jnp.float32)
    zn = jnp.zeros((CH, CW), jnp.float32)
    ones = jnp.ones((CH, CW), jnp.float32)

    h = _layernorm(x, norm_w, norm_b)
    aggp, cntp = _sc_agg_kernel()(h, src, dst, zc, zn, ones)
    return _combine(aggp, cntp, h, seg3, W_l, W_r, b_l, norm1_w, norm1_b)


# speedup vs baseline: 12.4077x; 2.0332x over previous
import functools

import jax
import jax.numpy as jnp
from jax import lax
from jax.experimental import pallas as pl
from jax.experimental.pallas import tpu as pltpu
from jax.experimental.pallas import tpu_sc as plsc

N_NODES = 10000
N_EDGES = 320000
D = 128
NUM_SEG = 64

NC = 2
NS = 16
NW = NC * NS
EPW = N_EDGES // NW
CH = 80
NCH = EPW // CH
NP = 10240
RPT = NP // NS
CW = 16

BLK = 1000
NBLK = N_NODES // BLK


def _ln_body(x_ref, w_ref, b_ref, o_ref):
    xv = x_ref[...]
    mu = jnp.mean(xv, axis=-1, keepdims=True)
    var = jnp.mean((xv - mu) ** 2, axis=-1, keepdims=True)
    o_ref[...] = (xv - mu) * lax.rsqrt(var + 1e-5) * w_ref[...] + b_ref[...]


def _layernorm(x, w, b):
    return pl.pallas_call(
        _ln_body,
        grid=(NBLK,),
        in_specs=[
            pl.BlockSpec((BLK, D), lambda i: (i, 0)),
            pl.BlockSpec((1, D), lambda i: (0, 0)),
            pl.BlockSpec((1, D), lambda i: (0, 0)),
        ],
        out_specs=pl.BlockSpec((BLK, D), lambda i: (i, 0)),
        out_shape=jax.ShapeDtypeStruct((N_NODES, D), jnp.float32),
    )(x, w.reshape(1, D), b.reshape(1, D))


def _sc_agg_body(h_hbm, src3_hbm, dst3_hbm, zc_hbm, zn_hbm, ones_hbm,
                 agg_out, cnt_out, agg_sh, cnt_sh,
                 sidx, didx, rows0, rows1, ones_v,
                 semi0, semi1, semi2, semi3, semg0, semg1):
    c = lax.axis_index("c")
    s = lax.axis_index("s")
    wid = c * NS + s
    r0 = s * RPT
    rowsb = (rows0, rows1)
    semi = (semi0, semi1, semi2, semi3)
    semg = (semg0, semg1)

    def fetch_idx(ci, b):
        pltpu.async_copy(src3_hbm.at[wid, ci], sidx.at[b], semi[b])
        pltpu.async_copy(dst3_hbm.at[wid, ci], didx.at[b], semi[b])

    def wait_idx(ci, b):
        pltpu.make_async_copy(src3_hbm.at[wid, ci], sidx.at[b], semi[b]).wait()
        pltpu.make_async_copy(dst3_hbm.at[wid, ci], didx.at[b], semi[b]).wait()

    def start_gather(b, rb):
        pltpu.async_copy(h_hbm.at[sidx.at[b]], rowsb[rb], semg[rb])

    def wait_gather(b, rb):
        pltpu.make_async_copy(h_hbm.at[sidx.at[b]], rowsb[rb], semg[rb]).wait()

    def scatter(b, rb):
        pltpu.sync_copy(rowsb[rb], agg_sh.at[didx.at[b]], add=True)
        pltpu.sync_copy(ones_v, cnt_sh.at[didx.at[b]], add=True)

    pltpu.sync_copy(zc_hbm, rows0)
    pltpu.sync_copy(zn_hbm, ones_v)
    for j in range(RPT // CH):
        pltpu.sync_copy(rows0, agg_sh.at[pl.ds(r0 + j * CH, CH)])
        pltpu.sync_copy(ones_v, cnt_sh.at[pl.ds(r0 + j * CH, CH)])
    pltpu.sync_copy(ones_hbm, ones_v)
    plsc.subcore_barrier()

    for x in range(4):
        fetch_idx(x, x)
    wait_idx(0, 0)
    start_gather(0, 0)
    wait_idx(1, 1)
    start_gather(1, 1)

    def step(x, off, do_fetch, do_gather):
        b = off % 4
        rb = off % 2
        wait_gather(b, rb)
        scatter(b, rb)
        if do_fetch:
            fetch_idx(x + 4, b)
        if do_gather:
            wait_idx(x + 2, (off + 2) % 4)
            start_gather((off + 2) % 4, rb)

    def body(m, carry):
        x0 = 4 * m
        for off in range(4):
            step(x0 + off, off, True, True)
        return carry

    lax.fori_loop(0, (NCH - 5) // 4, body, 0)
    base = NCH - 5
    step(base + 0, 0, True, True)
    step(base + 1, 1, False, True)
    step(base + 2, 2, False, True)
    step(base + 3, 3, False, False)
    step(base + 4, 0, False, False)
    plsc.subcore_barrier()
    for j in range(RPT // CH):
        pltpu.sync_copy(agg_sh.at[pl.ds(r0 + j * CH, CH)], rows0)
        pltpu.sync_copy(rows0, agg_out.at[c, pl.ds(r0 + j * CH, CH)])
        pltpu.sync_copy(cnt_sh.at[pl.ds(r0 + j * CH, CH)], ones_v)
        pltpu.sync_copy(ones_v, cnt_out.at[c, pl.ds(r0 + j * CH, CH)])


@functools.cache
def _sc_agg_kernel():
    return pl.kernel(
        _sc_agg_body,
        out_type=[
            jax.ShapeDtypeStruct((NC, NP, D), jnp.float32),
            jax.ShapeDtypeStruct((NC, NP, CW), jnp.float32),
        ],
        mesh=plsc.VectorSubcoreMesh(core_axis_name="c", subcore_axis_name="s",
                                    num_cores=NC, num_subcores=NS),
        scratch_types=[
            pltpu.VMEM_SHARED((NP, D), jnp.float32),
            pltpu.VMEM_SHARED((NP, CW), jnp.float32),
            pltpu.VMEM((4, CH), jnp.int32),
            pltpu.VMEM((4, CH), jnp.int32),
            pltpu.VMEM((CH, D), jnp.float32),
            pltpu.VMEM((CH, D), jnp.float32),
            pltpu.VMEM((CH, CW), jnp.float32),
            pltpu.SemaphoreType.DMA,
            pltpu.SemaphoreType.DMA,
            pltpu.SemaphoreType.DMA,
            pltpu.SemaphoreType.DMA,
            pltpu.SemaphoreType.DMA,
            pltpu.SemaphoreType.DMA,
        ],
        compiler_params=pltpu.CompilerParams(use_tc_tiling_on_sc=False),
    )


def _combine_body(aggp_ref, cntp_ref, h_ref, seg_ref, wl_ref, wr_ref, bl_ref,
                  n1w_ref, n1b_ref, o_ref, acc, cacc):
    i = pl.program_id(0)
    agg = aggp_ref[0] + aggp_ref[1]
    cnt = cntp_ref[0, :, 0:1] + cntp_ref[1, :, 0:1]
    mean_agg = agg / jnp.maximum(cnt, 1.0)
    h = h_ref[...]
    h2 = (
        lax.dot_general(mean_agg, wl_ref[...], (((1,), (1,)), ((), ())),
                        preferred_element_type=jnp.float32)
        + bl_ref[...]
        + lax.dot_general(h, wr_ref[...], (((1,), (1,)), ((), ())),
                          preferred_element_type=jnp.float32)
    )
    h2 = jnp.maximum(h2, 0.0)
    mu = jnp.mean(h2, axis=-1, keepdims=True)
    var = jnp.mean((h2 - mu) ** 2, axis=-1, keepdims=True)
    h2 = (h2 - mu) * lax.rsqrt(var + 1e-5) * n1w_ref[...] + n1b_ref[...]

    seg = seg_ref[0, 0, :]
    onehot = (seg[:, None]
              == lax.broadcasted_iota(jnp.int32, (BLK, NUM_SEG), 1)
              ).astype(jnp.float32)

    @pl.when(i == 0)
    def _():
        acc[...] = jnp.zeros_like(acc)
        cacc[...] = jnp.zeros_like(cacc)

    acc[...] += lax.dot_general(onehot, h2, (((0,), (0,)), ((), ())),
                                preferred_element_type=jnp.float32)
    cacc[...] += lax.dot_general(onehot, jnp.ones_like(h2),
                                 (((0,), (0,)), ((), ())),
                                 preferred_element_type=jnp.float32)

    @pl.when(i == NBLK - 1)
    def _():
        o_ref[...] = acc[...] / jnp.maximum(cacc[...], 1.0)


def _combine(aggp, cntp, h, seg3, W_l, W_r, b_l, n1w, n1b):
    return pl.pallas_call(
        _combine_body,
        grid=(NBLK,),
        in_specs=[
            pl.BlockSpec((NC, BLK, D), lambda i: (0, i, 0)),
            pl.BlockSpec((NC, BLK, CW), lambda i: (0, i, 0)),
            pl.BlockSpec((BLK, D), lambda i: (i, 0)),
            pl.BlockSpec((1, 1, BLK), lambda i: (i, 0, 0)),
            pl.BlockSpec((D, D), lambda i: (0, 0)),
            pl.BlockSpec((D, D), lambda i: (0, 0)),
            pl.BlockSpec((1, D), lambda i: (0, 0)),
            pl.BlockSpec((1, D), lambda i: (0, 0)),
            pl.BlockSpec((1, D), lambda i: (0, 0)),
        ],
        out_specs=pl.BlockSpec((NUM_SEG, D), lambda i: (0, 0)),
        out_shape=jax.ShapeDtypeStruct((NUM_SEG, D), jnp.float32),
        scratch_shapes=[
            pltpu.VMEM((NUM_SEG, D), jnp.float32),
            pltpu.VMEM((NUM_SEG, D), jnp.float32),
        ],
    )(aggp, cntp, h, seg3, W_l, W_r, b_l.reshape(1, D), n1w.reshape(1, D),
      n1b.reshape(1, D))


def kernel(x, edge_patch, superpixel_attri, W_l, b_l, W_r,
           norm_w, norm_b, norm1_w, norm1_b):
    src3 = edge_patch[0].astype(jnp.int32).reshape(NW, NCH, CH)
    dst3 = edge_patch[1].astype(jnp.int32).reshape(NW, NCH, CH)
    seg3 = superpixel_attri.astype(jnp.int32).reshape(NBLK, 1, BLK)
    zc = jnp.zeros((CH, D), jnp.float32)
    zn = jnp.zeros((CH, CW), jnp.float32)
    ones = jnp.ones((CH, CW), jnp.float32)

    h = _layernorm(x, norm_w, norm_b)
    aggp, cntp = _sc_agg_kernel()(h, src3, dst3, zc, zn, ones)
    return _combine(aggp, cntp, h, seg3, W_l, W_r, b_l, norm1_w, norm1_b)
